# baseline (speedup 1.0000x reference)
import functools

import jax
import jax.numpy as jnp
from jax import lax
from jax.experimental import pallas as pl
from jax.experimental.pallas import tpu as pltpu
from jax.experimental.pallas import tpu_sc as plsc

_B, _S, _D = 4, 4096, 1024
_NC, _NS = 2, 16
_NW = _NC * _NS
_ROWS_PER_W = _S // _NS
_CHUNK_ROWS = 32
_CHUNK = _CHUNK_ROWS * _D
_N_CHUNKS = _ROWS_PER_W // _CHUNK_ROWS
_XSZ = _S * _D
_STEPS = _N_CHUNKS * _B
_LN = 3


def _sc_body(x_hbm, pe_hbm, out_hbm, x0, x1, x2,
             lds0, lds1, lds2, sts0, sts1, sts2):
    @pl.when(lax.axis_index("c") == 0)
    def _core0_only():
        _do_copy(x_hbm, out_hbm,
                 (x0, x1, x2), (lds0, lds1, lds2), (sts0, sts1, sts2))


def _do_copy(x_hbm, out_hbm, x_bufs, ld_sems, st_sems):
    wid = lax.axis_index("s")
    base = wid * (_ROWS_PER_W * _D)

    def x_off(t):
        c, b = divmod(t, _B)
        return pl.multiple_of(b * _XSZ + base + c * _CHUNK, _CHUNK)

    def start_load(t):
        return pltpu.async_copy(
            x_hbm.at[pl.ds(x_off(t), _CHUNK)], x_bufs[t % _LN], ld_sems[t % _LN])

    def start_store(t):
        return pltpu.async_copy(
            x_bufs[t % _LN], out_hbm.at[pl.ds(x_off(t), _CHUNK)], st_sems[t % _LN])

    ld = [start_load(0), start_load(1), start_load(2)]
    st = [None, None, None]

    for t in range(_STEPS):
        ld[t % _LN].wait()
        st[t % _LN] = start_store(t)
        if t + _LN < _STEPS:
            st[t % _LN].wait()
            ld[t % _LN] = start_load(t + _LN)

    for k in range(_LN):
        if st[k] is not None:
            st[k].wait()


_sc_add = functools.partial(
    pl.kernel,
    mesh=plsc.VectorSubcoreMesh(core_axis_name="c", subcore_axis_name="s"),
    out_type=jax.ShapeDtypeStruct((_B * _S * _D,), jnp.float32),
    scratch_types=[
        pltpu.VMEM((_CHUNK,), jnp.float32),
        pltpu.VMEM((_CHUNK,), jnp.float32),
        pltpu.VMEM((_CHUNK,), jnp.float32),
        pltpu.SemaphoreType.DMA,
        pltpu.SemaphoreType.DMA,
        pltpu.SemaphoreType.DMA,
        pltpu.SemaphoreType.DMA,
        pltpu.SemaphoreType.DMA,
        pltpu.SemaphoreType.DMA,
    ],
)(_sc_body)


def kernel(x, pe):
    out = _sc_add(x.reshape(-1), pe.reshape(-1))
    return out.reshape(x.shape)


# speedup vs baseline: 4.5332x; 4.5332x over previous
import jax
import jax.numpy as jnp
from jax.experimental import pallas as pl


_S_BLK = 256


def _add_body(x_ref, pe_ref, o_ref):
    o_ref[...] = x_ref[...] + pe_ref[...][None, :, :]


def kernel(x, pe):
    B, S, D = x.shape
    grid = (S // _S_BLK,)
    return pl.pallas_call(
        _add_body,
        grid=grid,
        in_specs=[
            pl.BlockSpec((B, _S_BLK, D), lambda i: (0, i, 0)),
            pl.BlockSpec((_S_BLK, D), lambda i: (i, 0)),
        ],
        out_specs=pl.BlockSpec((B, _S_BLK, D), lambda i: (0, i, 0)),
        out_shape=jax.ShapeDtypeStruct((B, S, D), x.dtype),
    )(x, pe)


# speedup vs baseline: 4.6928x; 1.0352x over previous
import jax
import jax.numpy as jnp
from jax.experimental import pallas as pl


_S_BLK = 2048


def _add_body(x_ref, pe_ref, o_ref):
    o_ref[...] = x_ref[...] + pe_ref[...][None, :, :]


def kernel(x, pe):
    B, S, D = x.shape
    grid = (S // _S_BLK, B)
    return pl.pallas_call(
        _add_body,
        grid=grid,
        in_specs=[
            pl.BlockSpec((1, _S_BLK, D), lambda j, b: (b, j, 0)),
            pl.BlockSpec((_S_BLK, D), lambda j, b: (j, 0)),
        ],
        out_specs=pl.BlockSpec((1, _S_BLK, D), lambda j, b: (b, j, 0)),
        out_shape=jax.ShapeDtypeStruct((B, S, D), x.dtype),
    )(x, pe)
